# baseline (speedup 1.0000x reference)
import functools

import jax
import jax.numpy as jnp
import numpy as np
from jax import lax
from jax.experimental import pallas as pl
from jax.experimental.pallas import tpu as pltpu
from jax.experimental.pallas import tpu_sc as plsc


_B, _C, _P, _NC1, _KL = 4, 256, 1024, 21, 32


def _sc_downsample_body(lab_hbm, out_hbm, out_v, sem):
    wid = lax.axis_index("s") * 2 + lax.axis_index("c")
    t16 = lax.broadcasted_iota(jnp.int32, (16,), 0)
    copies = []
    for s in range(8):
        q = wid * 128 + s * 16 + t16
        idx = ((q >> 10) << 18) + (((q >> 5) & 31) << 13) + ((q & 31) << 4)
        copies.append(
            pltpu.async_copy(lab_hbm.at[idx], out_v.at[pl.ds(s * 16, 16)],
                             sem))
    for c in copies:
        c.wait()
    pltpu.sync_copy(out_v, out_hbm.at[pl.ds(wid * 128, 128)])


def _sc_downsample(labels1d):
    mesh = plsc.VectorSubcoreMesh(core_axis_name="c", subcore_axis_name="s")
    f = functools.partial(
        pl.kernel,
        out_type=jax.ShapeDtypeStruct((_B * _P, ), jnp.int32),
        mesh=mesh,
        scratch_types=[
            pltpu.VMEM((128,), jnp.int32),
            pltpu.SemaphoreType.DMA,
        ],
    )(_sc_downsample_body)
    return f(labels1d)


def _loss_kernel(ld_ref, feat_ref, protoT_ref, out_ref):
    f32 = jnp.float32
    ld = ld_ref[...]
    feat = feat_ref[...]
    protoT = protoT_ref[...]

    kio = jax.lax.broadcasted_iota(jnp.int32, (_B, _P, _KL), 2)
    oh = (ld[:, :, None] == kio).astype(f32)
    n_f = jnp.sum(oh, axis=1)

    pio_r = jax.lax.broadcasted_iota(jnp.int32, (_P, _P), 0)
    pio_c = jax.lax.broadcasted_iota(jnp.int32, (_P, _P), 1)
    LT = (pio_c < pio_r).astype(f32)
    rank_rows = []
    rs_list = []
    for b in range(_B):
        cums_b = jax.lax.dot_general(
            LT, oh[b], (((1,), (0,)), ((), ())),
            preferred_element_type=f32,
            precision=jax.lax.Precision.HIGHEST)
        rank_rows.append(jnp.sum(cums_b * oh[b], axis=1)[None, :])
        rs_list.append(jax.lax.dot_general(
            feat[b], oh[b], (((1,), (0,)), ((), ())),
            preferred_element_type=f32,
            precision=jax.lax.Precision.HIGHEST)[None])
    rank = jnp.concatenate(rank_rows, axis=0)
    RS = jnp.concatenate(rs_list, axis=0)

    g = jnp.sum(feat * feat, axis=1)
    ssq_all = jnp.sum(jnp.sum(g[:, :, None] * oh, axis=1), axis=0,
                      keepdims=True)

    n0, n1, n2, n3 = n_f[0:1], n_f[1:2], n_f[2:3], n_f[3:4]
    S_f = jnp.concatenate(
        [jnp.zeros_like(n0), n0, n0 + n1, n0 + n1 + n2], axis=0)
    N_f = n0 + n1 + n2 + n3

    cCf = jax.lax.broadcasted_iota(jnp.int32, (1, _C), 1).astype(f32)
    jjf = jax.lax.broadcasted_iota(jnp.int32, (_C, 1), 0).astype(f32)
    lane = jax.lax.broadcasted_iota(jnp.int32, (1, _KL), 1)

    def class_body(cl, carry):
        sums_accT = carry
        sel = (lane == cl).astype(f32)
        n_b = jnp.sum(n_f * sel, axis=1, keepdims=True)
        S_b = jnp.sum(S_f * sel, axis=1, keepdims=True)
        Ncl = jnp.sum(N_f * sel, axis=1, keepdims=True)
        Nsafe = jnp.maximum(Ncl, 1.0)
        rsel = jnp.sum(RS * sel[:, None, :], axis=2)
        x0 = _C * S_b + cCf * n_b
        q = jnp.floor(x0 / Nsafe)
        q = q + ((q + 1.0) * Nsafe <= x0).astype(f32)
        q = q - (q * Nsafe > x0).astype(f32)
        split = jnp.minimum((q + 1.0) * Nsafe - x0, n_b)
        rnkm = jnp.where(ld == cl, rank, 3.0e7)
        m = (rnkm[:, None, :] < split[:, :, None]).astype(f32)
        A = jnp.sum(feat * m, axis=2)
        acc = None
        for b in range(_B):
            sel0 = (q[b:b + 1] == jjf).astype(f32)
            sel1 = (q[b:b + 1] + 1.0 == jjf).astype(f32)
            part = jnp.sum(sel0 * A[b:b + 1] +
                           sel1 * (rsel[b:b + 1] - A[b:b + 1]), axis=1,
                           keepdims=True)
            acc = part if acc is None else acc + part
        return sums_accT + acc * sel

    sums_accT0 = jnp.zeros((_C, _KL), f32)
    sums_accT = jax.lax.fori_loop(1, _NC1, class_body, sums_accT0)

    sumsT = sums_accT[:, 1:_NC1]
    ssq_red = ssq_all[:, 1:_NC1]
    Nred = N_f[:, 1:_NC1]
    Nredsafe = jnp.maximum(Nred, 1.0)
    presf = Nred > 0.0
    Kp = jnp.sum(presf.astype(f32), axis=1, keepdims=True)
    Ksafe = jnp.maximum(Kp, 1.0)

    protoT_red = protoT[:, 1:_NC1]
    dots = jnp.sum(protoT_red * sumsT, axis=0, keepdims=True)
    psq = jnp.sum(protoT_red * protoT_red, axis=0, keepdims=True)
    mse = (ssq_red - 2.0 * dots + Nredsafe * psq) / (_C * Nredsafe)

    loss_fc = jnp.zeros((1, 1), f32)
    for i in range(_NC1 - 1):
        loss_fc = jnp.where(presf[:, i:i + 1],
                            (loss_fc + mse[:, i:i + 1]) / Ksafe, loss_fc)

    flmT = jnp.where(presf, sumsT / Nredsafe, 0.0)
    G = jax.lax.dot_general(flmT, flmT, (((0,), (0,)), ((), ())),
                            preferred_element_type=f32,
                            precision=jax.lax.Precision.HIGHEST)
    eio_r = jax.lax.broadcasted_iota(jnp.int32, (_NC1 - 1, _NC1 - 1), 0)
    eio_c = jax.lax.broadcasted_iota(jnp.int32, (_NC1 - 1, _NC1 - 1), 1)
    eyeb = eio_r == eio_c
    eyef = eyeb.astype(f32)
    diag = jnp.sum(G * eyef, axis=1, keepdims=True)
    diagT = jnp.sum(G * eyef, axis=0, keepdims=True)
    sq = diag + diagT - 2.0 * G
    row20 = jax.lax.broadcasted_iota(jnp.int32, (_NC1 - 1, 1), 0)
    NredCol = jnp.sum(N_f * ((row20 + 1) == lane).astype(f32),
                      axis=1, keepdims=True)
    presCol = NredCol > 0.0
    pair = (presCol & presf) & (~eyeb)
    sq_safe = jnp.where(pair, sq, 1.0)
    inv = 1.0 / jnp.sqrt(sq_safe)
    offd = pair.astype(f32)
    denom = jnp.sum(jnp.sum(offd, axis=1, keepdims=True), axis=0, keepdims=True)
    lsep = jnp.sum(jnp.sum(inv * offd, axis=1, keepdims=True),
                   axis=0, keepdims=True) / jnp.maximum(denom, 1.0)
    lsep = jnp.where(jnp.isnan(lsep), 0.0, lsep)
    loss_sep = jnp.where(Kp > 1.0, lsep, jnp.zeros((1, 1), f32))

    out_ref[...] = loss_fc + loss_sep


def kernel(labels, features_old, features, outputs_old, outputs, prototypes,
           num_class, num_old_class, num_new_class, epoch, train_step,
           len_epoch):
    B, C, h, w = features.shape
    H, W = labels.shape[1], labels.shape[2]
    ld = _sc_downsample(labels.astype(jnp.int32).reshape(B * H * W))
    ld = ld.reshape(B, h * w)
    feat = features.reshape(B, C, h * w)
    out = pl.pallas_call(
        _loss_kernel,
        out_shape=jax.ShapeDtypeStruct((1, 1), jnp.float32),
    )(ld, feat, prototypes.T)
    return out[0, 0]


# speedup vs baseline: 1.4221x; 1.4221x over previous
import functools

import jax
import jax.numpy as jnp
import numpy as np
from jax import lax
from jax.experimental import pallas as pl
from jax.experimental.pallas import tpu as pltpu
from jax.experimental.pallas import tpu_sc as plsc


_B, _C, _P, _NC1, _KL = 4, 256, 1024, 21, 32


def _sc_downsample_body(lab_hbm, out_hbm, idx_v, out_v, sem):
    wid = lax.axis_index("s") * 2 + lax.axis_index("c")
    t16 = lax.broadcasted_iota(jnp.int32, (16,), 0)
    for s in range(8):
        q = wid * 128 + s * 16 + t16
        idx_v[pl.ds(s * 16, 16)] = (((q >> 10) << 18) +
                                    (((q >> 5) & 31) << 13) + ((q & 31) << 4))
    pltpu.async_copy(lab_hbm.at[idx_v], out_v, sem).wait()
    pltpu.sync_copy(out_v, out_hbm.at[pl.ds(wid * 128, 128)])


def _sc_downsample(labels1d):
    mesh = plsc.VectorSubcoreMesh(core_axis_name="c", subcore_axis_name="s")
    f = functools.partial(
        pl.kernel,
        out_type=jax.ShapeDtypeStruct((_B * _P, ), jnp.int32),
        mesh=mesh,
        scratch_types=[
            pltpu.VMEM((128,), jnp.int32),
            pltpu.VMEM((128,), jnp.int32),
            pltpu.SemaphoreType.DMA,
        ],
    )(_sc_downsample_body)
    return f(labels1d)


def _loss_kernel(ld_ref, feat_ref, proto_ref, out_ref):
    f32 = jnp.float32
    HI = jax.lax.Precision.HIGHEST
    ld = ld_ref[...]
    feat = feat_ref[...]
    proto = proto_ref[...]

    kio = jax.lax.broadcasted_iota(jnp.int32, (_B, _P, _KL), 2)
    oh = (ld[:, :, None] == kio).astype(f32)
    n_f = jnp.sum(oh, axis=1)

    pio_r = jax.lax.broadcasted_iota(jnp.int32, (_P, _P), 0)
    pio_c = jax.lax.broadcasted_iota(jnp.int32, (_P, _P), 1)
    LT = (pio_c < pio_r).astype(f32)
    rank_rows = []
    rs_list = []
    for b in range(_B):
        cums_b = jax.lax.dot_general(
            LT, oh[b], (((1,), (0,)), ((), ())),
            preferred_element_type=f32)
        rank_rows.append(jnp.sum(cums_b * oh[b], axis=1)[None, :])
        rs_list.append(jax.lax.dot_general(
            feat[b], oh[b], (((1,), (0,)), ((), ())),
            preferred_element_type=f32, precision=HI)[None])
    rank = jnp.concatenate(rank_rows, axis=0)
    RS = jnp.concatenate(rs_list, axis=0)

    g = jnp.sum(feat * feat, axis=1)
    ssq_all = jnp.sum(jnp.sum(g[:, :, None] * oh, axis=1), axis=0,
                      keepdims=True)

    n0, n1, n2, n3 = n_f[0:1], n_f[1:2], n_f[2:3], n_f[3:4]
    S_f = jnp.concatenate(
        [jnp.zeros_like(n0), n0, n0 + n1, n0 + n1 + n2], axis=0)
    N_f = n0 + n1 + n2 + n3
    Nsafe_row = jnp.maximum(N_f, 1.0)

    cCcol = jax.lax.broadcasted_iota(jnp.int32, (_C, 1), 0).astype(f32)
    kio_col = jax.lax.broadcasted_iota(jnp.int32, (_KL, 1), 0)
    q0_list, A_list = [], []
    for b in range(_B):
        n_row = n_f[b:b + 1]
        S_row = S_f[b:b + 1]
        x0 = _C * S_row + cCcol * n_row
        q0 = jnp.floor(x0 / Nsafe_row)
        q0 = q0 + ((q0 + 1.0) * Nsafe_row <= x0).astype(f32)
        q0 = q0 - (q0 * Nsafe_row > x0).astype(f32)
        split = jnp.minimum((q0 + 1.0) * Nsafe_row - x0, n_row)
        ohT_b = (ld[b:b + 1] == kio_col).astype(f32)
        split_g = jax.lax.dot_general(
            split, ohT_b, (((1,), (0,)), ((), ())),
            preferred_element_type=f32, precision=HI)
        contrib = jnp.where(rank[b:b + 1] < split_g, feat[b], 0.0)
        A_list.append(jax.lax.dot_general(
            contrib, oh[b], (((1,), (0,)), ((), ())),
            preferred_element_type=f32, precision=HI)[None])
        q0_list.append(q0[None])
    A_st = jnp.concatenate(A_list, axis=0)
    q0_st = jnp.concatenate(q0_list, axis=0)

    j3 = jax.lax.broadcasted_iota(jnp.int32, (1, 1, _C), 2).astype(f32)
    sums_rows = []
    for cl in range(1, _NC1):
        q0_sl = q0_st[:, :, cl:cl + 1]
        A_sl = A_st[:, :, cl:cl + 1]
        R_sl = RS[:, :, cl:cl + 1]
        t = (jnp.where(q0_sl == j3, A_sl, 0.0) +
             jnp.where(q0_sl + 1.0 == j3, R_sl - A_sl, 0.0))
        sums_rows.append(jnp.sum(jnp.sum(t, axis=0), axis=0, keepdims=True))
    sums_all = jnp.concatenate(sums_rows, axis=0)

    lane = jax.lax.broadcasted_iota(jnp.int32, (1, _KL), 1)
    row20 = jax.lax.broadcasted_iota(jnp.int32, (_NC1 - 1, 1), 0)
    NredCol = jnp.sum(N_f * ((row20 + 1) == lane).astype(f32),
                      axis=1, keepdims=True)
    ssqCol = jnp.sum(ssq_all * ((row20 + 1) == lane).astype(f32),
                     axis=1, keepdims=True)
    Nredsafe = jnp.maximum(NredCol, 1.0)
    presf = NredCol > 0.0
    presfT = N_f[:, 1:_NC1] > 0.0
    Kp = jnp.sum(presf.astype(f32), axis=0, keepdims=True)
    Ksafe = jnp.maximum(Kp, 1.0)

    proto_red = proto[1:_NC1]
    dots = jnp.sum(proto_red * sums_all, axis=1, keepdims=True)
    psq = jnp.sum(proto_red * proto_red, axis=1, keepdims=True)
    mse = (ssqCol - 2.0 * dots + Nredsafe * psq) / (_C * Nredsafe)

    loss_fc = jnp.zeros((1, 1), f32)
    for i in range(_NC1 - 1):
        loss_fc = jnp.where(presf[i:i + 1], (loss_fc + mse[i:i + 1]) / Ksafe,
                            loss_fc)

    flm = jnp.where(presf, sums_all / Nredsafe, 0.0)
    G = jax.lax.dot_general(flm, flm, (((1,), (1,)), ((), ())),
                            preferred_element_type=f32,
                            precision=HI)
    eio_r = jax.lax.broadcasted_iota(jnp.int32, (_NC1 - 1, _NC1 - 1), 0)
    eio_c = jax.lax.broadcasted_iota(jnp.int32, (_NC1 - 1, _NC1 - 1), 1)
    eyeb = eio_r == eio_c
    eyef = eyeb.astype(f32)
    diag = jnp.sum(G * eyef, axis=1, keepdims=True)
    diagT = jnp.sum(G * eyef, axis=0, keepdims=True)
    sq = diag + diagT - 2.0 * G
    pair = (presf & presfT) & (~eyeb)
    sq_safe = jnp.where(pair, sq, 1.0)
    inv = 1.0 / jnp.sqrt(sq_safe)
    offd = pair.astype(f32)
    denom = jnp.sum(jnp.sum(offd, axis=1, keepdims=True), axis=0, keepdims=True)
    lsep = jnp.sum(jnp.sum(inv * offd, axis=1, keepdims=True),
                   axis=0, keepdims=True) / jnp.maximum(denom, 1.0)
    lsep = jnp.where(jnp.isnan(lsep), 0.0, lsep)
    loss_sep = jnp.where(Kp > 1.0, lsep, jnp.zeros((1, 1), f32))

    out_ref[...] = loss_fc + loss_sep


def kernel(labels, features_old, features, outputs_old, outputs, prototypes,
           num_class, num_old_class, num_new_class, epoch, train_step,
           len_epoch):
    B, C, h, w = features.shape
    H, W = labels.shape[1], labels.shape[2]
    ld = _sc_downsample(labels.astype(jnp.int32).reshape(B * H * W))
    ld = ld.reshape(B, h * w)
    feat = features.reshape(B, C, h * w)
    out = pl.pallas_call(
        _loss_kernel,
        out_shape=jax.ShapeDtypeStruct((1, 1), jnp.float32),
    )(ld, feat, prototypes)
    return out[0, 0]
